# Initial kernel scaffold; baseline (speedup 1.0000x reference)
import functools

import jax
import jax.numpy as jnp
from jax import lax
from jax.experimental import pallas as pl
from jax.experimental.pallas import tpu as pltpu
from jax.experimental.pallas import tpu_sc as plsc

N = 10000
E = 320000
D = 128

NC = 2
NS = 16
NW = NC * NS
EPW = E // NW
CHUNK = 128
NCHUNK = 80
EPW_PAD = NCHUNK * CHUNK
PAD = EPW_PAD - EPW
NPAD = N + 8
RPT = N // NS


def _sc_agg(with_deg: bool):
    mesh = plsc.VectorSubcoreMesh(core_axis_name="c", subcore_axis_name="s")
    out_type = [jax.ShapeDtypeStruct((NC * N, D), jnp.float32)]
    if with_deg:
        out_type.append(jax.ShapeDtypeStruct((NC * N, 16), jnp.float32))
    scratch = [
        pltpu.VMEM((NCHUNK, 2, CHUNK), jnp.int32),
        pltpu.VMEM((CHUNK, D), jnp.float32),
        pltpu.VMEM((CHUNK, D), jnp.float32),
        pltpu.VMEM((25, D), jnp.float32),
        pltpu.VMEM((125, 16), jnp.float32),
        pltpu.VMEM((125, D), jnp.float32),
        pltpu.VMEM((CHUNK, 16), jnp.float32),
        pltpu.SemaphoreType.DMA,
        pltpu.SemaphoreType.DMA,
        pltpu.SemaphoreType.DMA,
        pltpu.VMEM_SHARED((NPAD, D), jnp.float32),
        pltpu.VMEM_SHARED((NPAD, 16), jnp.float32),
    ]

    def body(table, packed, *refs):
        if with_deg:
            out, deg_out = refs[0], refs[1]
            rest = refs[2:]
        else:
            out = refs[0]
            rest = refs[1:]
        (idx_all, buf_a, buf_b, zb, zd, stage, ones_v,
         sem_a, sem_b, sem_z, agg_sh, deg_sh) = rest

        cid = lax.axis_index("c")
        sid = lax.axis_index("s")
        wid = cid * NS + sid
        s0 = sid * RPT

        zrow = jnp.zeros((16,), jnp.float32)
        orow = jnp.ones((16,), jnp.float32)
        for r in range(25):
            for k in range(D // 16):
                zb[r, pl.ds(k * 16, 16)] = zrow
        for r in range(125):
            zd[r, :] = zrow
        if with_deg:
            for r in range(CHUNK):
                ones_v[r, :] = orow

        zcopies = []
        for i in range(25):
            zcopies.append(pltpu.async_copy(
                zb, agg_sh.at[pl.ds(s0 + i * 25, 25)], sem_z))
        if with_deg:
            for i in range(5):
                zcopies.append(pltpu.async_copy(
                    zd, deg_sh.at[pl.ds(s0 + i * 125, 125)], sem_z))
        for c in zcopies:
            c.wait()
        plsc.subcore_barrier()

        pltpu.sync_copy(packed.at[pl.ds(wid * NCHUNK, NCHUNK)], idx_all)

        def scatter(buf, g):
            pltpu.sync_copy(buf, agg_sh.at[idx_all.at[g, 1]], add=True)
            if with_deg:
                pltpu.sync_copy(ones_v, deg_sh.at[idx_all.at[g, 1]], add=True)

        pltpu.async_copy(table.at[idx_all.at[0, 0]], buf_a, sem_a)

        def pair(i, carry):
            g0 = 2 * i
            pltpu.async_copy(table.at[idx_all.at[g0 + 1, 0]], buf_b, sem_b)
            pltpu.make_async_copy(table.at[pl.ds(0, CHUNK)], buf_a, sem_a).wait()
            scatter(buf_a, g0)
            pltpu.async_copy(table.at[idx_all.at[g0 + 2, 0]], buf_a, sem_a)
            pltpu.make_async_copy(table.at[pl.ds(0, CHUNK)], buf_b, sem_b).wait()
            scatter(buf_b, g0 + 1)
            return carry

        lax.fori_loop(0, NCHUNK // 2 - 1, pair, 0)
        pltpu.async_copy(table.at[idx_all.at[NCHUNK - 1, 0]], buf_b, sem_b)
        pltpu.make_async_copy(table.at[pl.ds(0, CHUNK)], buf_a, sem_a).wait()
        scatter(buf_a, NCHUNK - 2)
        pltpu.make_async_copy(table.at[pl.ds(0, CHUNK)], buf_b, sem_b).wait()
        scatter(buf_b, NCHUNK - 1)

        plsc.subcore_barrier()

        for i in range(5):
            pltpu.sync_copy(agg_sh.at[pl.ds(s0 + i * 125, 125)], stage)
            pltpu.sync_copy(stage, out.at[pl.ds(cid * N + s0 + i * 125, 125)])
        if with_deg:
            for i in range(5):
                pltpu.sync_copy(deg_sh.at[pl.ds(s0 + i * 125, 125)], zd)
                pltpu.sync_copy(zd, deg_out.at[pl.ds(cid * N + s0 + i * 125, 125)])

    return pl.kernel(body, out_type=out_type, mesh=mesh,
                     scratch_types=scratch)


_sc_agg_deg_kernel = _sc_agg(True)
_sc_agg_kernel = _sc_agg(False)

_R = 1000


def _tc_layer_body(relu, x_ref, p0_ref, p1_ref, d0_ref, d1_ref,
                   w_ref, b_ref, o_ref):
    d = d0_ref[:, 0:1] + d1_ref[:, 0:1]
    inv = 1.0 / jnp.maximum(d, 1.0)
    agg = (p0_ref[...] + p1_ref[...]) * inv
    w = w_ref[...]
    out = lax.dot_general(x_ref[...], w[:, :D], (((1,), (1,)), ((), ())),
                          preferred_element_type=jnp.float32,
                          precision=lax.Precision.HIGHEST)
    out += lax.dot_general(agg, w[:, D:], (((1,), (1,)), ((), ())),
                           preferred_element_type=jnp.float32,
                           precision=lax.Precision.HIGHEST)
    out += b_ref[...]
    if relu:
        out = jnp.maximum(out, 0.0)
    o_ref[...] = out


def _tc_layer(x, p_flat, deg_flat, w, b, relu):
    grid = N // _R
    nb = N // _R
    body = functools.partial(_tc_layer_body, relu)
    return pl.pallas_call(
        body,
        grid=(grid,),
        in_specs=[
            pl.BlockSpec((_R, D), lambda i: (i, 0)),
            pl.BlockSpec((_R, D), lambda i: (i, 0)),
            pl.BlockSpec((_R, D), lambda i, _nb=nb: (i + _nb, 0)),
            pl.BlockSpec((_R, 16), lambda i: (i, 0)),
            pl.BlockSpec((_R, 16), lambda i, _nb=nb: (i + _nb, 0)),
            pl.BlockSpec((D, 2 * D), lambda i: (0, 0)),
            pl.BlockSpec((1, D), lambda i: (0, 0)),
        ],
        out_specs=pl.BlockSpec((_R, D), lambda i: (i, 0)),
        out_shape=jax.ShapeDtypeStruct((N, D), jnp.float32),
    )(x, p_flat, p_flat, deg_flat, deg_flat, w, b)


def kernel(x, edge_index, W1, b1, W2, b2):
    row = edge_index[0]
    col = edge_index[1]
    colp = jnp.concatenate(
        [col.reshape(NW, EPW),
         jnp.zeros((NW, PAD), jnp.int32)], axis=1).reshape(NW, NCHUNK, CHUNK)
    rowp = jnp.concatenate(
        [row.reshape(NW, EPW),
         jnp.full((NW, PAD), N, jnp.int32)], axis=1).reshape(NW, NCHUNK, CHUNK)
    packed = jnp.stack([colp, rowp], axis=2).reshape(NW * NCHUNK, 2, CHUNK)

    b1r = b1.reshape(1, D)
    b2r = b2.reshape(1, D)

    p1, deg = _sc_agg_deg_kernel(x, packed)
    h = _tc_layer(x, p1, deg, W1, b1r, True)
    p2 = _sc_agg_kernel(h, packed)
    return _tc_layer(h, p2, deg, W2, b2r, False)


# speedup vs baseline: 4.9300x; 4.9300x over previous
import functools

import jax
import jax.numpy as jnp
from jax import lax
from jax.experimental import pallas as pl
from jax.experimental.pallas import tpu as pltpu
from jax.experimental.pallas import tpu_sc as plsc

N = 10000
E = 320000
D = 128
HD = D // 2

NC = 2
NS = 16
EPW = E // NS
CHUNK = 128
NCHUNK = 158
PAD = NCHUNK * CHUNK - EPW
NACC = 10240
ZPT = NACC // NS
RO = 624
DCH = NCHUNK // 2

_SC_PARAMS = pltpu.CompilerParams(use_tc_tiling_on_sc=False)


def _make_sc_agg():
    mesh = plsc.VectorSubcoreMesh(core_axis_name="c", subcore_axis_name="s")
    scratch = [
        pltpu.VMEM((NCHUNK, 3, CHUNK), jnp.int32),
        pltpu.VMEM((CHUNK, HD), jnp.float32),
        pltpu.VMEM((CHUNK, HD), jnp.float32),
        pltpu.VMEM((64, HD), jnp.float32),
        pltpu.SemaphoreType.DMA,
        pltpu.SemaphoreType.DMA,
        pltpu.SemaphoreType.DMA,
        pltpu.VMEM_SHARED((NACC, HD), jnp.float32),
    ]

    def body(table, packed, out, idx_all, buf_a, buf_b, zb,
             sem_a, sem_b, sem_z, agg_sh):
        cid = lax.axis_index("c")
        sid = lax.axis_index("s")
        z0 = sid * ZPT
        r0 = sid * RO

        zrow = jnp.zeros((16,), jnp.float32)
        for r in range(64):
            for k in range(HD // 16):
                zb[r, pl.ds(k * 16, 16)] = zrow

        zcopies = [pltpu.async_copy(zb, agg_sh.at[pl.ds(z0 + i * 64, 64)],
                                    sem_z) for i in range(10)]
        for c in zcopies:
            c.wait()
        plsc.subcore_barrier()

        pltpu.sync_copy(packed.at[pl.ds(sid * NCHUNK, NCHUNK)], idx_all)

        def gather(buf, sem, g):
            pltpu.async_copy(table.at[idx_all.at[g, cid]], buf, sem)

        def drain(buf, sem):
            pltpu.make_async_copy(table.at[pl.ds(0, CHUNK)], buf, sem).wait()

        def scatter(buf, g):
            pltpu.sync_copy(buf, agg_sh.at[idx_all.at[g, 2]], add=True)

        gather(buf_a, sem_a, 0)

        def pair(i, carry):
            g0 = 2 * i
            gather(buf_b, sem_b, g0 + 1)
            drain(buf_a, sem_a)
            scatter(buf_a, g0)
            gather(buf_a, sem_a, g0 + 2)
            drain(buf_b, sem_b)
            scatter(buf_b, g0 + 1)
            return carry

        lax.fori_loop(0, (NCHUNK - 2) // 2, pair, 0)
        gather(buf_b, sem_b, NCHUNK - 1)
        drain(buf_a, sem_a)
        scatter(buf_a, NCHUNK - 2)
        drain(buf_b, sem_b)
        scatter(buf_b, NCHUNK - 1)

        plsc.subcore_barrier()

        for i in range(3):
            pltpu.sync_copy(agg_sh.at[pl.ds(r0 + i * 208, 208)],
                            out.at[pl.ds(cid * N + r0 + i * 208, 208)])

        @pl.when(sid == NS - 1)
        def _tail():
            pltpu.sync_copy(agg_sh.at[pl.ds(NS * RO, 16)],
                            out.at[pl.ds(cid * N + NS * RO, 16)])

    return pl.kernel(body,
                     out_type=jax.ShapeDtypeStruct((NC * N, HD), jnp.float32),
                     mesh=mesh, scratch_types=scratch,
                     compiler_params=_SC_PARAMS)


def _make_sc_deg():
    mesh = plsc.VectorSubcoreMesh(core_axis_name="c", subcore_axis_name="s")
    scratch = [
        pltpu.VMEM((DCH, 3, CHUNK), jnp.int32),
        pltpu.VMEM((CHUNK, 16), jnp.float32),
        pltpu.VMEM((64, 16), jnp.float32),
        pltpu.SemaphoreType.DMA,
        pltpu.VMEM_SHARED((NACC, 16), jnp.float32),
    ]

    def body(packed, deg_out, idx_all, ones_v, zd, sem_z, deg_sh):
        cid = lax.axis_index("c")
        sid = lax.axis_index("s")
        z0 = sid * ZPT
        r0 = sid * RO

        zrow = jnp.zeros((16,), jnp.float32)
        orow = jnp.ones((16,), jnp.float32)
        for r in range(64):
            zd[r, :] = zrow
        for r in range(CHUNK):
            ones_v[r, :] = orow

        zcopies = [pltpu.async_copy(zd, deg_sh.at[pl.ds(z0 + i * 64, 64)],
                                    sem_z) for i in range(10)]
        for c in zcopies:
            c.wait()
        plsc.subcore_barrier()

        pltpu.sync_copy(packed.at[pl.ds(sid * NCHUNK + cid * DCH, DCH)],
                        idx_all)

        def step(g, carry):
            pltpu.sync_copy(ones_v, deg_sh.at[idx_all.at[g, 2]], add=True)
            return carry

        lax.fori_loop(0, DCH, step, 0)
        plsc.subcore_barrier()

        for i in range(3):
            pltpu.sync_copy(deg_sh.at[pl.ds(r0 + i * 208, 208)],
                            deg_out.at[pl.ds(cid * N + r0 + i * 208, 208)])

        @pl.when(sid == NS - 1)
        def _tail():
            pltpu.sync_copy(deg_sh.at[pl.ds(NS * RO, 16)],
                            deg_out.at[pl.ds(cid * N + NS * RO, 16)])

    return pl.kernel(body,
                     out_type=jax.ShapeDtypeStruct((NC * N, 16), jnp.float32),
                     mesh=mesh, scratch_types=scratch,
                     compiler_params=_SC_PARAMS)


_sc_agg_kernel = _make_sc_agg()
_sc_deg_kernel = _make_sc_deg()

_R = 1000


def _tc_layer_body(relu, x_ref, p0_ref, p1_ref, d0_ref, d1_ref,
                   w_ref, b_ref, o_ref):
    d = d0_ref[:, 0:1] + d1_ref[:, 0:1]
    inv = 1.0 / jnp.maximum(d, 1.0)
    w = w_ref[...]
    dn = (((1,), (1,)), ((), ()))
    out = lax.dot_general(x_ref[...], w[:, :D], dn,
                          preferred_element_type=jnp.float32,
                          precision=lax.Precision.HIGHEST)
    out += lax.dot_general(p0_ref[...] * inv, w[:, D:D + HD], dn,
                           preferred_element_type=jnp.float32,
                           precision=lax.Precision.HIGHEST)
    out += lax.dot_general(p1_ref[...] * inv, w[:, D + HD:], dn,
                           preferred_element_type=jnp.float32,
                           precision=lax.Precision.HIGHEST)
    out += b_ref[...]
    if relu:
        out = jnp.maximum(out, 0.0)
    o_ref[...] = out


def _tc_layer(x, p_flat, deg_flat, w, b, relu):
    grid = N // _R
    nb = N // _R
    body = functools.partial(_tc_layer_body, relu)
    return pl.pallas_call(
        body,
        grid=(grid,),
        in_specs=[
            pl.BlockSpec((_R, D), lambda i: (i, 0)),
            pl.BlockSpec((_R, HD), lambda i: (i, 0)),
            pl.BlockSpec((_R, HD), lambda i, _nb=nb: (i + _nb, 0)),
            pl.BlockSpec((_R, 16), lambda i: (i, 0)),
            pl.BlockSpec((_R, 16), lambda i, _nb=nb: (i + _nb, 0)),
            pl.BlockSpec((D, 2 * D), lambda i: (0, 0)),
            pl.BlockSpec((1, D), lambda i: (0, 0)),
        ],
        out_specs=pl.BlockSpec((_R, D), lambda i: (i, 0)),
        out_shape=jax.ShapeDtypeStruct((N, D), jnp.float32),
    )(x, p_flat, p_flat, deg_flat, deg_flat, w, b)


def kernel(x, edge_index, W1, b1, W2, b2):
    row = edge_index[0]
    col = edge_index[1]
    colp = jnp.concatenate(
        [col.reshape(NS, EPW),
         jnp.zeros((NS, PAD), jnp.int32)], axis=1).reshape(NS, NCHUNK, CHUNK)
    rowp = jnp.concatenate(
        [row.reshape(NS, EPW),
         jnp.full((NS, PAD), N, jnp.int32)], axis=1).reshape(NS, NCHUNK, CHUNK)
    packed = jnp.stack([2 * colp, 2 * colp + 1, rowp],
                       axis=2).reshape(NS * NCHUNK, 3, CHUNK)

    b1r = b1.reshape(1, D)
    b2r = b2.reshape(1, D)

    deg = _sc_deg_kernel(packed)
    p1 = _sc_agg_kernel(x.reshape(2 * N, HD), packed)
    h = _tc_layer(x, p1, deg, W1, b1r, True)
    p2 = _sc_agg_kernel(h.reshape(2 * N, HD), packed)
    return _tc_layer(h, p2, deg, W2, b2r, False)
